# Initial kernel scaffold; baseline (speedup 1.0000x reference)
import jax
import jax.numpy as jnp
from jax.experimental import pallas as pl


def kernel(x, edge_index, drug_index, label, c1_W1, c1_b1, c1_W2, c1_b2, c2_W1, c2_b1, c2_W2, c2_b2, c3_W1, c3_b1, c3_W2, c3_b2, P1, P2):
    raise NotImplementedError("write your pallas kernel here")


# speedup vs baseline: 3.9286x; 3.9286x over previous
import functools

import jax
import jax.numpy as jnp
from jax import lax
from jax.experimental import pallas as pl
from jax.experimental.pallas import tpu as pltpu
from jax.experimental.pallas import tpu_sc as plsc

_EPS = 0.01
_N = 10000
_E = 160000
_C = 128
_NTILES = 16
_EB = 80
_EPT = _E // _NTILES
_NB = _EPT // _EB
_NPAD = 10240
_RPT = _NPAD // _NTILES


@functools.lru_cache(maxsize=None)
def _make_agg(nc):
    mesh = plsc.VectorSubcoreMesh(core_axis_name="c", subcore_axis_name="s")

    def body(*refs):
        tables = refs[0:nc]
        src_hbm, dst_hbm, zeros_hbm = refs[nc:nc + 3]
        outs = refs[nc + 3:2 * nc + 3]
        src_v, dst_v, rows_v, acc, sem = refs[2 * nc + 3:]

        cid = lax.axis_index("c")
        sid = lax.axis_index("s")

        pltpu.sync_copy(src_hbm.at[sid], src_v)
        pltpu.sync_copy(dst_hbm.at[sid], dst_v)

        for c in range(nc):
            @pl.when(cid == (c % 2))
            def _round(c=c):
                plsc.subcore_barrier()
                pltpu.sync_copy(zeros_hbm, acc.at[pl.ds(sid * _RPT, _RPT)])
                plsc.subcore_barrier()

                def step(j, carry):
                    pltpu.async_copy(
                        tables[c].at[src_v.at[j]], rows_v, sem).wait()
                    pltpu.sync_copy(rows_v, acc.at[dst_v.at[j]], add=True)
                    return carry

                lax.fori_loop(0, _NB, step, 0)
                plsc.subcore_barrier()
                pltpu.sync_copy(acc.at[pl.ds(sid * _RPT, _RPT)],
                                outs[c].at[pl.ds(sid * _RPT, _RPT)])

    return pl.kernel(
        body,
        out_type=[jax.ShapeDtypeStruct((_NPAD, _C), jnp.float32)
                  for _ in range(nc)],
        mesh=mesh,
        scratch_types=[
            pltpu.VMEM((_NB, _EB), jnp.int32),
            pltpu.VMEM((_NB, _EB), jnp.int32),
            pltpu.VMEM((_EB, _C), jnp.float32),
            pltpu.VMEM_SHARED((_NPAD, _C), jnp.float32),
            pltpu.SemaphoreType.DMA,
        ],
    )


@functools.lru_cache(maxsize=None)
def _make_pair_gather():
    mesh = plsc.VectorSubcoreMesh(core_axis_name="c", subcore_axis_name="s")

    def body(t0, t1, idx_hbm, o0, o1, idx_v, rows_v, sem):
        cid = lax.axis_index("c")
        sid = lax.axis_index("s")
        pltpu.sync_copy(idx_hbm.at[sid], idx_v)
        for c, (tab, out) in enumerate(((t0, o0), (t1, o1))):
            @pl.when(cid == c)
            def _chunk(tab=tab, out=out):
                pltpu.async_copy(tab.at[idx_v], rows_v, sem).wait()
                pltpu.sync_copy(rows_v, out.at[pl.ds(sid * 128, 128)])

    return pl.kernel(
        body,
        out_type=[jax.ShapeDtypeStruct((2048, _C), jnp.float32)
                  for _ in range(2)],
        mesh=mesh,
        scratch_types=[
            pltpu.VMEM((128,), jnp.int32),
            pltpu.VMEM((128, _C), jnp.float32),
            pltpu.SemaphoreType.DMA,
        ],
    )


def _mlp(h_chunks, xn_chunks, W1, b1, W2, b2):
    kc = len(h_chunks)
    K, J = W1.shape
    O = W2.shape[1]
    oc = O // _C
    R = 1000
    grid = (_N // R,)

    def body(*refs):
        h_refs = refs[0:kc]
        xn_refs = refs[kc:2 * kc]
        w1_ref, b1_ref, w2_ref, b2_ref = refs[2 * kc:2 * kc + 4]
        out_refs = refs[2 * kc + 4:]
        hin = jnp.concatenate(
            [(1.0 + _EPS) * h_refs[c][...] + xn_refs[c][...]
             for c in range(kc)], axis=1)
        z = jnp.dot(hin, w1_ref[...], preferred_element_type=jnp.float32)
        z = jnp.maximum(z + b1_ref[...], 0.0)
        o = jnp.dot(z, w2_ref[...], preferred_element_type=jnp.float32)
        o = jnp.maximum(o + b2_ref[...], 0.0)
        for c in range(oc):
            out_refs[c][...] = o[:, c * _C:(c + 1) * _C]

    row_spec = pl.BlockSpec((R, _C), lambda r: (r, 0))
    full = pl.BlockSpec
    return pl.pallas_call(
        body,
        grid=grid,
        in_specs=(
            [row_spec] * kc + [row_spec] * kc
            + [full((K, J), lambda r: (0, 0)),
               full((1, J), lambda r: (0, 0)),
               full((J, O), lambda r: (0, 0)),
               full((1, O), lambda r: (0, 0))]
        ),
        out_specs=[row_spec] * oc,
        out_shape=[jax.ShapeDtypeStruct((_N, _C), jnp.float32)
                   for _ in range(oc)],
    )(*h_chunks, *xn_chunks, W1, b1.reshape(1, -1), W2, b2.reshape(1, -1))


def _decode(pair_chunks, P1, P2):
    def body(pr0, pr1, p1_ref, p2_ref, out_ref):
        ap = None
        bp = None
        for c, pr in enumerate((pr0, pr1)):
            w = p1_ref[c * _C:(c + 1) * _C, :]
            pa = jnp.dot(pr[0:1024, :], w, preferred_element_type=jnp.float32)
            pb = jnp.dot(pr[1024:2048, :], w,
                         preferred_element_type=jnp.float32)
            ap = pa if ap is None else ap + pa
            bp = pb if bp is None else bp + pb
        t2 = jnp.dot(ap, p2_ref[...], preferred_element_type=jnp.float32)
        out_ref[...] = jnp.sum(t2 * bp, axis=1, keepdims=True)

    return pl.pallas_call(
        body,
        out_shape=jax.ShapeDtypeStruct((1024, 1), jnp.float32),
    )(*pair_chunks, P1, P2)


def _chunked(a):
    n, d = a.shape
    return [a[:, i * _C:(i + 1) * _C] for i in range(d // _C)]


def kernel(x, edge_index, drug_index, label,
           c1_W1, c1_b1, c1_W2, c1_b2,
           c2_W1, c2_b1, c2_W2, c2_b2,
           c3_W1, c3_b1, c3_W2, c3_b2,
           P1, P2):
    src = edge_index[0].reshape(_NTILES, _NB, _EB)
    dst = edge_index[1].reshape(_NTILES, _NB, _EB)
    zeros = jnp.zeros((_RPT, _C), jnp.float32)

    xc = _chunked(x)
    agg2 = _make_agg(2)
    agg4 = _make_agg(4)

    xn1 = agg2(*xc, src, dst, zeros)
    h1 = _mlp(xc, xn1, c1_W1, c1_b1, c1_W2, c1_b2)
    xn2 = agg4(*h1, src, dst, zeros)
    h2 = _mlp(h1, xn2, c2_W1, c2_b1, c2_W2, c2_b2)
    xn3 = agg4(*h2, src, dst, zeros)
    h3 = _mlp(h2, xn3, c3_W1, c3_b1, c3_W2, c3_b2)

    di = drug_index.reshape(-1, 2)
    pidx = jnp.concatenate([di[:, 0] - 1, di[:, 1] - 1]).reshape(_NTILES, 128)
    pairs = _make_pair_gather()(*h3, pidx)
    return _decode(pairs, P1, P2)


# speedup vs baseline: 6.3485x; 1.6160x over previous
import functools

import jax
import jax.numpy as jnp
from jax import lax
from jax.experimental import pallas as pl
from jax.experimental.pallas import tpu as pltpu
from jax.experimental.pallas import tpu_sc as plsc

_EPS = 0.01
_N = 10000
_E = 160000
_C = 128
_NTILES = 16
_EB = 80
_EPT = _E // _NTILES
_NB = _EPT // _EB
_NPAD = 10240
_RPT = _NPAD // _NTILES


@functools.lru_cache(maxsize=None)
def _make_agg(nc):
    mesh = plsc.VectorSubcoreMesh(core_axis_name="c", subcore_axis_name="s")

    def body(*refs):
        tables = refs[0:nc]
        src_hbm, dst_hbm, zeros_hbm = refs[nc:nc + 3]
        outs = refs[nc + 3:2 * nc + 3]
        src_v, dst_v, rows0, rows1, acc, sem0, sem1 = refs[2 * nc + 3:]

        cid = lax.axis_index("c")
        sid = lax.axis_index("s")

        pltpu.sync_copy(src_hbm.at[pl.ds(sid * _EPT, _EPT)], src_v)
        pltpu.sync_copy(dst_hbm.at[sid], dst_v)

        for c in range(nc):
            @pl.when(cid == (c % 2))
            def _round(c=c):
                plsc.subcore_barrier()
                pltpu.sync_copy(zeros_hbm, acc.at[pl.ds(sid * _RPT, _RPT)])
                plsc.subcore_barrier()

                pltpu.async_copy(
                    tables[c].at[src_v.at[pl.ds(0, _EB)]], rows0, sem0)
                pltpu.async_copy(
                    tables[c].at[src_v.at[pl.ds(_EB, _EB)]], rows1, sem1)

                def lane(j, buf, sem):
                    pltpu.make_async_copy(
                        tables[c].at[src_v.at[pl.ds(j * _EB, _EB)]],
                        buf, sem).wait()
                    pltpu.sync_copy(buf, acc.at[dst_v.at[j]], add=True)

                    @pl.when(j + 2 < _NB)
                    def _():
                        pltpu.async_copy(
                            tables[c].at[src_v.at[pl.ds((j + 2) * _EB, _EB)]],
                            buf, sem)

                def step(j, carry):
                    @pl.when(j % 2 == 0)
                    def _():
                        lane(j, rows0, sem0)

                    @pl.when(j % 2 == 1)
                    def _():
                        lane(j, rows1, sem1)
                    return carry

                lax.fori_loop(0, _NB, step, 0)
                plsc.subcore_barrier()
                pltpu.sync_copy(acc.at[pl.ds(sid * _RPT, _RPT)],
                                outs[c].at[pl.ds(sid * _RPT, _RPT)])

    return pl.kernel(
        body,
        out_type=[jax.ShapeDtypeStruct((_NPAD, _C), jnp.float32)
                  for _ in range(nc)],
        mesh=mesh,
        scratch_types=[
            pltpu.VMEM((_EPT,), jnp.int32),
            pltpu.VMEM((_NB, _EB), jnp.int32),
            pltpu.VMEM((_EB, _C), jnp.float32),
            pltpu.VMEM((_EB, _C), jnp.float32),
            pltpu.VMEM_SHARED((_NPAD, _C), jnp.float32),
            pltpu.SemaphoreType.DMA,
            pltpu.SemaphoreType.DMA,
        ],
    )


@functools.lru_cache(maxsize=None)
def _make_pair_gather():
    mesh = plsc.VectorSubcoreMesh(core_axis_name="c", subcore_axis_name="s")

    def body(t0, t1, idx_hbm, o0, o1, idx_v, rows_v, sem):
        cid = lax.axis_index("c")
        sid = lax.axis_index("s")
        pltpu.sync_copy(idx_hbm.at[sid], idx_v)
        for c, (tab, out) in enumerate(((t0, o0), (t1, o1))):
            @pl.when(cid == c)
            def _chunk(tab=tab, out=out):
                pltpu.async_copy(tab.at[idx_v], rows_v, sem).wait()
                pltpu.sync_copy(rows_v, out.at[pl.ds(sid * 128, 128)])

    return pl.kernel(
        body,
        out_type=[jax.ShapeDtypeStruct((2048, _C), jnp.float32)
                  for _ in range(2)],
        mesh=mesh,
        scratch_types=[
            pltpu.VMEM((128,), jnp.int32),
            pltpu.VMEM((128, _C), jnp.float32),
            pltpu.SemaphoreType.DMA,
        ],
    )


def _mlp(h_chunks, xn_chunks, W1, b1, W2, b2):
    kc = len(h_chunks)
    K, J = W1.shape
    O = W2.shape[1]
    oc = O // _C
    R = 1000
    grid = (_N // R,)

    def body(*refs):
        h_refs = refs[0:kc]
        xn_refs = refs[kc:2 * kc]
        w1_ref, b1_ref, w2_ref, b2_ref = refs[2 * kc:2 * kc + 4]
        out_refs = refs[2 * kc + 4:]
        hin = jnp.concatenate(
            [(1.0 + _EPS) * h_refs[c][...] + xn_refs[c][...]
             for c in range(kc)], axis=1)
        z = jnp.dot(hin, w1_ref[...], preferred_element_type=jnp.float32)
        z = jnp.maximum(z + b1_ref[...], 0.0)
        o = jnp.dot(z, w2_ref[...], preferred_element_type=jnp.float32)
        o = jnp.maximum(o + b2_ref[...], 0.0)
        for c in range(oc):
            out_refs[c][...] = o[:, c * _C:(c + 1) * _C]

    row_spec = pl.BlockSpec((R, _C), lambda r: (r, 0))
    full = pl.BlockSpec
    return pl.pallas_call(
        body,
        grid=grid,
        in_specs=(
            [row_spec] * kc + [row_spec] * kc
            + [full((K, J), lambda r: (0, 0)),
               full((1, J), lambda r: (0, 0)),
               full((J, O), lambda r: (0, 0)),
               full((1, O), lambda r: (0, 0))]
        ),
        out_specs=[row_spec] * oc,
        out_shape=[jax.ShapeDtypeStruct((_N, _C), jnp.float32)
                   for _ in range(oc)],
    )(*h_chunks, *xn_chunks, W1, b1.reshape(1, -1), W2, b2.reshape(1, -1))


def _decode(pair_chunks, P1, P2):
    def body(pr0, pr1, p1_ref, p2_ref, out_ref):
        ap = None
        bp = None
        for c, pr in enumerate((pr0, pr1)):
            w = p1_ref[c * _C:(c + 1) * _C, :]
            pa = jnp.dot(pr[0:1024, :], w, preferred_element_type=jnp.float32)
            pb = jnp.dot(pr[1024:2048, :], w,
                         preferred_element_type=jnp.float32)
            ap = pa if ap is None else ap + pa
            bp = pb if bp is None else bp + pb
        t2 = jnp.dot(ap, p2_ref[...], preferred_element_type=jnp.float32)
        out_ref[...] = jnp.sum(t2 * bp, axis=1, keepdims=True)

    return pl.pallas_call(
        body,
        out_shape=jax.ShapeDtypeStruct((1024, 1), jnp.float32),
    )(*pair_chunks, P1, P2)


def _chunked(a):
    n, d = a.shape
    return [a[:, i * _C:(i + 1) * _C] for i in range(d // _C)]


def kernel(x, edge_index, drug_index, label,
           c1_W1, c1_b1, c1_W2, c1_b2,
           c2_W1, c2_b1, c2_W2, c2_b2,
           c3_W1, c3_b1, c3_W2, c3_b2,
           P1, P2):
    src = edge_index[0]
    dst = edge_index[1].reshape(_NTILES, _NB, _EB)
    zeros = jnp.zeros((_RPT, _C), jnp.float32)

    xc = _chunked(x)
    agg2 = _make_agg(2)
    agg4 = _make_agg(4)

    xn1 = agg2(*xc, src, dst, zeros)
    h1 = _mlp(xc, xn1, c1_W1, c1_b1, c1_W2, c1_b2)
    xn2 = agg4(*h1, src, dst, zeros)
    h2 = _mlp(h1, xn2, c2_W1, c2_b1, c2_W2, c2_b2)
    xn3 = agg4(*h2, src, dst, zeros)
    h3 = _mlp(h2, xn3, c3_W1, c3_b1, c3_W2, c3_b2)

    di = drug_index.reshape(-1, 2)
    pidx = jnp.concatenate([di[:, 0] - 1, di[:, 1] - 1]).reshape(_NTILES, 128)
    pairs = _make_pair_gather()(*h3, pidx)
    return _decode(pairs, P1, P2)
